# baseline (speedup 1.0000x reference)
import jax
import jax.numpy as jnp
from jax.experimental import pallas as pl
from jax.experimental.pallas import tpu as pltpu

NUM_Q_HEADS = 32
NUM_KV_HEADS = 8
HEAD_DIM = 128
GQA = NUM_Q_HEADS // NUM_KV_HEADS
SCALE = HEAD_DIM ** -0.5
NUM_BLOCKS = 2048
BLOCK_SIZE = 16
B = 16
MAX_BLOCKS_PER_SEQ = 128
MAX_LEN = MAX_BLOCKS_PER_SEQ * BLOCK_SIZE
FUSED = NUM_KV_HEADS * HEAD_DIM
CHUNK = 16
CHUNK_TOK = CHUNK * BLOCK_SIZE
MAX_CHUNKS = MAX_BLOCKS_PER_SEQ // CHUNK
NBUF = 4
AHEAD = NBUF - 1
NEG_INF = float("-inf")


def _attn_body(nb_ref, bt_ref, sl_ref,
               qbd_ref, k_new_ref, v_new_ref, mask_ref, excl_ref, cnt_ref,
               kc_hbm, vc_hbm,
               out_ref,
               m_ref, l_ref, acc_ref, kcat_ref, vcat_ref, sem_ref):
    b = pl.program_id(0)
    nb = nb_ref[b]
    nchunks = (nb + CHUNK - 1) // CHUNK

    def _issue(c):
        slot = jax.lax.rem(c, NBUF)
        for j in range(CHUNK):
            pos = jnp.minimum(c * CHUNK + j, nb - 1)
            pb = bt_ref[b, pos]
            pltpu.make_async_copy(
                kc_hbm.at[pb],
                kcat_ref.at[slot, pl.ds(j * BLOCK_SIZE, BLOCK_SIZE)],
                sem_ref.at[slot]).start()
            pltpu.make_async_copy(
                vc_hbm.at[pb],
                vcat_ref.at[slot, pl.ds(j * BLOCK_SIZE, BLOCK_SIZE)],
                sem_ref.at[slot]).start()

    m_ref[...] = jnp.full_like(m_ref, NEG_INF)
    l_ref[...] = jnp.zeros_like(l_ref)
    acc_ref[...] = jnp.zeros_like(acc_ref)

    jax.lax.fori_loop(0, jnp.minimum(AHEAD, nchunks),
                      lambda c, _: (_issue(c), 0)[1], 0)

    def _chunk_body(c, _):
        @pl.when(c + AHEAD < nchunks)
        def _issue_ahead():
            _issue(c + AHEAD)

        slot = jax.lax.rem(c, NBUF)
        for j in range(CHUNK):
            pltpu.make_async_copy(
                kc_hbm.at[bt_ref[b, 0]],
                kcat_ref.at[slot, pl.ds(j * BLOCK_SIZE, BLOCK_SIZE)],
                sem_ref.at[slot]).wait()
            pltpu.make_async_copy(
                vc_hbm.at[bt_ref[b, 0]],
                vcat_ref.at[slot, pl.ds(j * BLOCK_SIZE, BLOCK_SIZE)],
                sem_ref.at[slot]).wait()

        kc = kcat_ref[slot]
        vc = vcat_ref[slot]
        s = jax.lax.dot_general(
            qbd_ref[0], kc, (((1,), (1,)), ((), ())),
            preferred_element_type=jnp.float32) * SCALE
        rem = sl_ref[b] - c * CHUNK_TOK
        lane = jax.lax.broadcasted_iota(jnp.int32, (NUM_Q_HEADS, CHUNK_TOK), 1)
        ex = excl_ref[0, 0, pl.ds(c * CHUNK_TOK, CHUNK_TOK)]
        keep = jnp.logical_and(lane < rem, (ex < 0.5)[None, :])
        s = jnp.where(keep, s, NEG_INF)
        m_old = m_ref[...]
        m_new = jnp.maximum(m_old, jnp.max(s, axis=1, keepdims=True))
        alpha = jnp.exp(m_old - m_new)
        p = jnp.exp(s - m_new)
        l_ref[...] = alpha * l_ref[...] + jnp.sum(p, axis=1, keepdims=True)
        pv = jax.lax.dot_general(
            p, vc, (((1,), (0,)), ((), ())),
            preferred_element_type=jnp.float32)
        acc_ref[...] = alpha * acc_ref[...] + pv
        m_ref[...] = m_new
        return 0

    jax.lax.fori_loop(0, nchunks, _chunk_body, 0)

    cnt = cnt_ref[0, 0]
    s_f = jax.lax.dot_general(
        qbd_ref[0], k_new_ref[...], (((1,), (1,)), ((), ())),
        preferred_element_type=jnp.float32) * SCALE
    s_f = jnp.where((cnt > 0.5)[None, :], s_f, NEG_INF)
    m_old = m_ref[...]
    m_new = jnp.maximum(m_old, jnp.max(s_f, axis=1, keepdims=True))
    alpha = jnp.exp(m_old - m_new)
    p_f = jnp.exp(s_f - m_new) * cnt[None, :]
    l_fin = alpha * l_ref[...] + jnp.sum(p_f, axis=1, keepdims=True)
    pv_f = jax.lax.dot_general(
        p_f, v_new_ref[...], (((1,), (0,)), ((), ())),
        preferred_element_type=jnp.float32)
    acc_fin = alpha * acc_ref[...] + pv_f

    a = acc_fin * mask_ref[...]
    o = a[:, 0:HEAD_DIM]
    for j in range(1, NUM_KV_HEADS):
        o = o + a[:, j * HEAD_DIM : (j + 1) * HEAD_DIM]
    out_ref[0] = o / l_fin


@jax.jit
def _paged_attn(q, k, v, k_cache, v_cache, slot_mapping, block_tables,
                seq_lens):
    nb = (seq_lens + BLOCK_SIZE - 1) // BLOCK_SIZE
    kc3 = k_cache.reshape(NUM_BLOCKS, BLOCK_SIZE, FUSED)
    vc3 = v_cache.reshape(NUM_BLOCKS, BLOCK_SIZE, FUSED)
    k2 = k.reshape(B, FUSED)
    v2 = v.reshape(B, FUSED)
    bd = (jnp.arange(FUSED)[None, :] // HEAD_DIM
          == jnp.arange(NUM_Q_HEADS)[:, None] // GQA)
    bd = bd.astype(jnp.float32)
    q_bd = jnp.tile(q, (1, 1, NUM_KV_HEADS)) * bd[None]

    slot_i32 = slot_mapping.astype(jnp.int32)
    slot_blk = slot_i32 // BLOCK_SIZE
    slot_off = slot_i32 % BLOCK_SIZE
    occ = block_tables[:, :, None] == slot_blk[None, None, :]
    hit_off = (slot_off[None, None, :, None]
               == jnp.arange(BLOCK_SIZE)[None, None, None, :])
    excl = jnp.any(occ[:, :, :, None] & hit_off, axis=2)
    excl = excl.reshape(B, 1, MAX_LEN).astype(jnp.float32)
    pos_tok = (jnp.arange(MAX_BLOCKS_PER_SEQ)[None, :, None] * BLOCK_SIZE
               + slot_off[None, None, :])
    in_range = pos_tok < seq_lens[:, None, None]
    cnt = jnp.sum((occ & in_range).astype(jnp.float32), axis=1)
    wi = jnp.arange(B)
    dup_later = jnp.any((slot_i32[None, :] == slot_i32[:, None])
                        & (wi[None, :] > wi[:, None]), axis=1)
    cnt = cnt * (~dup_later)[None, :].astype(jnp.float32)
    cnt = cnt.reshape(B, 1, B)

    grid_spec = pltpu.PrefetchScalarGridSpec(
        num_scalar_prefetch=3,
        grid=(B,),
        in_specs=[
            pl.BlockSpec((1, NUM_Q_HEADS, FUSED), lambda b, *_: (b, 0, 0)),
            pl.BlockSpec((B, FUSED), lambda b, *_: (0, 0)),
            pl.BlockSpec((B, FUSED), lambda b, *_: (0, 0)),
            pl.BlockSpec((NUM_Q_HEADS, FUSED), lambda b, *_: (0, 0)),
            pl.BlockSpec((1, 1, MAX_LEN), lambda b, *_: (b, 0, 0)),
            pl.BlockSpec((1, 1, B), lambda b, *_: (b, 0, 0)),
            pl.BlockSpec(memory_space=pltpu.MemorySpace.HBM),
            pl.BlockSpec(memory_space=pltpu.MemorySpace.HBM),
        ],
        out_specs=pl.BlockSpec((1, NUM_Q_HEADS, HEAD_DIM),
                               lambda b, *_: (b, 0, 0)),
        scratch_shapes=[
            pltpu.VMEM((NUM_Q_HEADS, 1), jnp.float32),
            pltpu.VMEM((NUM_Q_HEADS, 1), jnp.float32),
            pltpu.VMEM((NUM_Q_HEADS, FUSED), jnp.float32),
            pltpu.VMEM((NBUF, CHUNK_TOK, FUSED), jnp.float32),
            pltpu.VMEM((NBUF, CHUNK_TOK, FUSED), jnp.float32),
            pltpu.SemaphoreType.DMA((NBUF,)),
        ],
    )
    return pl.pallas_call(
        _attn_body,
        grid_spec=grid_spec,
        out_shape=jax.ShapeDtypeStruct((B, NUM_Q_HEADS, HEAD_DIM),
                                       jnp.float32),
        compiler_params=pltpu.CompilerParams(
            dimension_semantics=("arbitrary",)),
    )(nb, block_tables, seq_lens,
      q_bd, k2, v2, bd, excl, cnt, kc3, vc3)


def kernel(q, k, v, k_cache, v_cache, slot_mapping, block_tables, seq_lens,
           query_lens, is_prefill):
    del query_lens, is_prefill
    return _paged_attn(q, k, v, k_cache, v_cache, slot_mapping, block_tables,
                       seq_lens)
